# baseline (speedup 1.0000x reference)
import functools

import jax
import jax.numpy as jnp
from jax import lax
from jax.experimental import pallas as pl
from jax.experimental.pallas import tpu as pltpu
from jax.experimental.pallas import tpu_sc as plsc

N_NODES = 10000
N_EDGES = 320000
D = 128

NC = 2
NS = 16
NW = NC * NS
E_PER_W = N_EDGES // NW
CHUNK = 125
NCHUNK = E_PER_W // CHUNK
NBUF = 2
PHASE_A = 40
N_ACC = N_NODES
RPT = 624
TAIL = N_NODES - RPT * NS
TAIL_OFF = RPT * NS


def _sc_aggregate(h, src2, dst2, zeros):
    mesh = plsc.VectorSubcoreMesh(core_axis_name="c", subcore_axis_name="s")

    @functools.partial(
        pl.kernel,
        out_type=jax.ShapeDtypeStruct((NC, N_NODES, D), jnp.float32),
        mesh=mesh,
        scratch_types=[
            pltpu.VMEM((PHASE_A + 4, CHUNK), jnp.int32),
            pltpu.VMEM((PHASE_A + 4, CHUNK), jnp.int32),
            pltpu.VMEM((CHUNK, D), jnp.float32),
            pltpu.VMEM((CHUNK, D), jnp.float32),
            pltpu.VMEM_SHARED((N_ACC, D), jnp.float32),
            pltpu.SemaphoreType.DMA((NBUF,)),
            pltpu.SemaphoreType.DMA((NBUF,)),
        ],
    )
    def body(h_hbm, src_hbm, dst_hbm, z_hbm, out_hbm, src_v, dst_v, rows0, rows1,
             acc, gsem, ssem):
        rows = (rows0, rows1)
        cid = lax.axis_index("c")
        sid = lax.axis_index("s")
        wid = sid * NC + cid
        pltpu.sync_copy(z_hbm.at[pl.ds(sid * RPT, RPT)],
                        acc.at[pl.ds(sid * RPT, RPT)])

        @pl.when(sid == NS - 1)
        def _():
            pltpu.sync_copy(z_hbm.at[pl.ds(TAIL_OFF, TAIL)],
                            acc.at[pl.ds(TAIL_OFF, TAIL)])
        plsc.subcore_barrier()

        def g_start(r, b):
            pltpu.async_copy(h_hbm.at[src_v.at[r]], rows[b], gsem.at[b])

        def g_wait(r, b):
            pltpu.make_async_copy(h_hbm.at[src_v.at[r]], rows[b],
                                  gsem.at[b]).wait()

        def s_start(r, b):
            pltpu.async_copy(rows[b], acc.at[dst_v.at[r]], ssem.at[b],
                             add=True)

        def s_wait(r, b):
            pltpu.make_async_copy(rows[b], acc.at[dst_v.at[r]],
                                  ssem.at[b]).wait()

        def slot(r, b):
            g_start(r, b)
            g_wait(r, b)
            s_start(r, b)

        def phase(c0, n):
            pltpu.sync_copy(src_hbm.at[wid, pl.ds(c0, n)], src_v.at[pl.ds(0, n)])
            pltpu.sync_copy(dst_hbm.at[wid, pl.ds(c0, n)], dst_v.at[pl.ds(0, n)])
            pro = 2 + (n % 2)
            for t in range(pro):
                if t >= 2:
                    s_wait(t - 2, t % 2)
                slot(t, t % 2)

            def steady(i, carry):
                for k in (0, 1):
                    r = 2 * i + pro + k
                    b = (pro + k) % 2
                    s_wait(r - 2, b)
                    slot(r, b)
                return carry

            lax.fori_loop(0, (n - pro) // 2, steady, 0)
            s_wait(n - 2, (n - 2) % 2)
            s_wait(n - 1, (n - 1) % 2)

        phase(0, PHASE_A)
        phase(PHASE_A, NCHUNK - PHASE_A)
        plsc.subcore_barrier()
        pltpu.sync_copy(acc.at[pl.ds(sid * RPT, RPT)],
                        out_hbm.at[cid, pl.ds(sid * RPT, RPT)])

        @pl.when(sid == NS - 1)
        def _():
            pltpu.sync_copy(acc.at[pl.ds(TAIL_OFF, TAIL)],
                            out_hbm.at[cid, pl.ds(TAIL_OFF, TAIL)])

    return body(h, src2, dst2, zeros)


def _tc_layer(h, agg2, w_root, w_neigh, bias, do_relu):
    blk = 2000
    grid = N_NODES // blk

    def body(h_ref, a_ref, wr_ref, wn_ref, b_ref, o_ref):
        agg = a_ref[0] + a_ref[1]
        acc = jnp.dot(h_ref[...], wr_ref[...], preferred_element_type=jnp.float32)
        acc += jnp.dot(agg, wn_ref[...], preferred_element_type=jnp.float32)
        acc += b_ref[...]
        if do_relu:
            acc = jnp.maximum(acc, 0.0)
        o_ref[...] = acc

    return pl.pallas_call(
        body,
        out_shape=jax.ShapeDtypeStruct((N_NODES, D), jnp.float32),
        grid=(grid,),
        in_specs=[
            pl.BlockSpec((blk, D), lambda i: (i, 0)),
            pl.BlockSpec((NC, blk, D), lambda i: (0, i, 0)),
            pl.BlockSpec((D, D), lambda i: (0, 0)),
            pl.BlockSpec((D, D), lambda i: (0, 0)),
            pl.BlockSpec((1, D), lambda i: (0, 0)),
        ],
        out_specs=pl.BlockSpec((blk, D), lambda i: (i, 0)),
    )(h, agg2, w_root, w_neigh, bias)


def kernel(x, edge_index, W_root1, W_neigh1, b1, W_root2, W_neigh2, b2):
    src2 = edge_index[0].reshape(NW, NCHUNK, CHUNK)
    dst2 = edge_index[1].reshape(NW, NCHUNK, CHUNK)
    zeros = jnp.zeros((N_NODES, D), jnp.float32)
    b1r = b1.reshape(1, D)
    b2r = b2.reshape(1, D)

    agg1 = _sc_aggregate(x, src2, dst2, zeros)
    h1 = _tc_layer(x, agg1, W_root1, W_neigh1, b1r, True)
    agg2 = _sc_aggregate(h1, src2, dst2, zeros)
    return _tc_layer(h1, agg2, W_root2, W_neigh2, b2r, False)


# speedup vs baseline: 1.1622x; 1.1622x over previous
import functools

import jax
import jax.numpy as jnp
from jax import lax
from jax.experimental import pallas as pl
from jax.experimental.pallas import tpu as pltpu
from jax.experimental.pallas import tpu_sc as plsc

N_NODES = 10000
N_EDGES = 320000
D = 128

NC = 2
NS = 16
NW = NC * NS
E_PER_W = N_EDGES // NW
CHUNK = 125
NCHUNK = E_PER_W // CHUNK
NBUF = 2
PHASE_A = 40
N_ACC = N_NODES
RPT = 624
TAIL = N_NODES - RPT * NS
TAIL_OFF = RPT * NS


def _sc_aggregate(h, src2, dst2, zeros):
    mesh = plsc.VectorSubcoreMesh(core_axis_name="c", subcore_axis_name="s")

    @functools.partial(
        pl.kernel,
        out_type=jax.ShapeDtypeStruct((NC, N_NODES, D), jnp.float32),
        mesh=mesh,
        scratch_types=[
            pltpu.VMEM((PHASE_A + 4, CHUNK), jnp.int32),
            pltpu.VMEM((PHASE_A + 4, CHUNK), jnp.int32),
            pltpu.VMEM((CHUNK, D), jnp.float32),
            pltpu.VMEM((CHUNK, D), jnp.float32),
            pltpu.VMEM_SHARED((N_ACC, D), jnp.float32),
            pltpu.SemaphoreType.DMA((NBUF,)),
            pltpu.SemaphoreType.DMA((NBUF,)),
        ],
    )
    def body(h_hbm, src_hbm, dst_hbm, z_hbm, out_hbm, src_v, dst_v, rows0, rows1,
             acc, gsem, ssem):
        rows = (rows0, rows1)
        cid = lax.axis_index("c")
        sid = lax.axis_index("s")
        wid = sid * NC + cid
        pltpu.sync_copy(z_hbm.at[pl.ds(sid * RPT, RPT)],
                        acc.at[pl.ds(sid * RPT, RPT)])

        @pl.when(sid == NS - 1)
        def _():
            pltpu.sync_copy(z_hbm.at[pl.ds(TAIL_OFF, TAIL)],
                            acc.at[pl.ds(TAIL_OFF, TAIL)])
        plsc.subcore_barrier()

        def g_start(r, b):
            pltpu.async_copy(h_hbm.at[src_v.at[r]], rows[b], gsem.at[b])

        def g_wait(r, b):
            pltpu.make_async_copy(h_hbm.at[src_v.at[r]], rows[b],
                                  gsem.at[b]).wait()

        def s_start(r, b):
            pltpu.async_copy(rows[b], acc.at[dst_v.at[r]], ssem.at[b],
                             add=True)

        def s_wait(r, b):
            pltpu.make_async_copy(rows[b], acc.at[dst_v.at[r]],
                                  ssem.at[b]).wait()

        def uslot(r, b):
            s_wait(r - 1, 1 - b)
            g_start(r + 1, 1 - b)
            g_wait(r, b)
            s_start(r, b)

        def phase(c0, n):
            pltpu.sync_copy(src_hbm.at[wid, pl.ds(c0, n)], src_v.at[pl.ds(0, n)])
            pltpu.sync_copy(dst_hbm.at[wid, pl.ds(c0, n)], dst_v.at[pl.ds(0, n)])
            g_start(0, 0)
            g_start(1, 1)
            g_wait(0, 0)
            s_start(0, 0)
            uslot(1, 1)

            def steady(i, carry):
                for k in (0, 1):
                    uslot(2 * i + 2 + k, k)
                return carry

            lax.fori_loop(0, (n - 4) // 2, steady, 0)
            uslot(n - 2, (n - 2) % 2)
            g_wait(n - 1, (n - 1) % 2)
            s_start(n - 1, (n - 1) % 2)
            s_wait(n - 2, (n - 2) % 2)
            s_wait(n - 1, (n - 1) % 2)

        phase(0, PHASE_A)
        phase(PHASE_A, NCHUNK - PHASE_A)
        plsc.subcore_barrier()
        pltpu.sync_copy(acc.at[pl.ds(sid * RPT, RPT)],
                        out_hbm.at[cid, pl.ds(sid * RPT, RPT)])

        @pl.when(sid == NS - 1)
        def _():
            pltpu.sync_copy(acc.at[pl.ds(TAIL_OFF, TAIL)],
                            out_hbm.at[cid, pl.ds(TAIL_OFF, TAIL)])

    return body(h, src2, dst2, zeros)


def _tc_layer(h, agg2, w_root, w_neigh, bias, do_relu):
    blk = 2000
    grid = N_NODES // blk

    def body(h_ref, a_ref, wr_ref, wn_ref, b_ref, o_ref):
        agg = a_ref[0] + a_ref[1]
        acc = jnp.dot(h_ref[...], wr_ref[...], preferred_element_type=jnp.float32)
        acc += jnp.dot(agg, wn_ref[...], preferred_element_type=jnp.float32)
        acc += b_ref[...]
        if do_relu:
            acc = jnp.maximum(acc, 0.0)
        o_ref[...] = acc

    return pl.pallas_call(
        body,
        out_shape=jax.ShapeDtypeStruct((N_NODES, D), jnp.float32),
        grid=(grid,),
        in_specs=[
            pl.BlockSpec((blk, D), lambda i: (i, 0)),
            pl.BlockSpec((NC, blk, D), lambda i: (0, i, 0)),
            pl.BlockSpec((D, D), lambda i: (0, 0)),
            pl.BlockSpec((D, D), lambda i: (0, 0)),
            pl.BlockSpec((1, D), lambda i: (0, 0)),
        ],
        out_specs=pl.BlockSpec((blk, D), lambda i: (i, 0)),
    )(h, agg2, w_root, w_neigh, bias)


def kernel(x, edge_index, W_root1, W_neigh1, b1, W_root2, W_neigh2, b2):
    src2 = edge_index[0].reshape(NW, NCHUNK, CHUNK)
    dst2 = edge_index[1].reshape(NW, NCHUNK, CHUNK)
    zeros = jnp.zeros((N_NODES, D), jnp.float32)
    b1r = b1.reshape(1, D)
    b2r = b2.reshape(1, D)

    agg1 = _sc_aggregate(x, src2, dst2, zeros)
    h1 = _tc_layer(x, agg1, W_root1, W_neigh1, b1r, True)
    agg2 = _sc_aggregate(h1, src2, dst2, zeros)
    return _tc_layer(h1, agg2, W_root2, W_neigh2, b2r, False)
